# baseline (speedup 1.0000x reference)
import jax
import jax.numpy as jnp
from jax.experimental import pallas as pl
from jax.experimental.pallas import tpu as pltpu

_TOT = 32768
_D = 3
_NSEG = 16
_BS = 8192
_GRID = _TOT // _BS


def _fused(cu_ref, xt_ref, w1, b1, w2, b2, w3, b3,
           v1, c1, vp, c3, out_ref, acc_ref):
    i = pl.program_id(0)

    @pl.when(i == 0)
    def _init():
        acc_ref[...] = jnp.zeros_like(acc_ref)

    dn = (((0,), (0,)), ((), ()))
    zero = jnp.bfloat16(0.0)
    xt = xt_ref[0].astype(jnp.bfloat16)
    b1c = jnp.transpose(b1[...])
    b2c = jnp.transpose(b2[...])
    b3c = jnp.transpose(b3[...])
    h = jnp.maximum(jax.lax.dot_general(
        w1[...].astype(jnp.bfloat16), xt, dn,
        preferred_element_type=jnp.float32).astype(jnp.bfloat16)
        + b1c.astype(jnp.bfloat16), zero)
    h = jnp.maximum(jax.lax.dot_general(
        w2[...].astype(jnp.bfloat16), h, dn,
        preferred_element_type=jnp.float32).astype(jnp.bfloat16)
        + b2c.astype(jnp.bfloat16), zero)
    h = jnp.maximum(jax.lax.dot_general(
        w3[...].astype(jnp.bfloat16), h, dn,
        preferred_element_type=jnp.float32).astype(jnp.bfloat16)
        + b3c.astype(jnp.bfloat16), zero)

    iota_col = jax.lax.broadcasted_iota(jnp.int32, (_NSEG, 1), 0)
    cu_lo = jnp.zeros((_NSEG, 1), jnp.int32)
    cu_hi = jnp.zeros((_NSEG, 1), jnp.int32)
    for s in range(_NSEG):
        cu_lo = jnp.where(iota_col == s, cu_ref[s], cu_lo)
        cu_hi = jnp.where(iota_col == s, cu_ref[s + 1], cu_hi)
    rows = i * _BS + jax.lax.broadcasted_iota(jnp.int32, (_NSEG, _BS), 1)
    onehot_t = ((rows >= cu_lo) & (rows < cu_hi)).astype(jnp.bfloat16)
    acc_ref[...] += jax.lax.dot_general(
        h, onehot_t, (((1,), (1,)), ((), ())),
        preferred_element_type=jnp.float32)

    @pl.when(i == _GRID - 1)
    def _finish():
        iota_row = jax.lax.broadcasted_iota(jnp.int32, (1, _NSEG), 1)
        cnt = jnp.ones((1, _NSEG), jnp.float32)
        for s in range(_NSEG):
            c = (cu_ref[s + 1] - cu_ref[s]).astype(jnp.float32)
            cnt = jnp.where(iota_row == s, c, cnt)
        pooled = jnp.transpose(acc_ref[...] / jnp.maximum(cnt, 1.0))
        vp_arr = vp[...]
        o = jnp.maximum(jnp.dot(pooled.astype(jnp.bfloat16),
                                v1[...].astype(jnp.bfloat16),
                                preferred_element_type=jnp.float32) + c1[...],
                        0.0).astype(jnp.bfloat16)
        o = jnp.maximum(jnp.dot(o, vp_arr[0:128],
                                preferred_element_type=jnp.float32)
                        + vp_arr[256:257].astype(jnp.float32),
                        0.0).astype(jnp.bfloat16)
        out_ref[...] = jnp.dot(o, vp_arr[128:256],
                               preferred_element_type=jnp.float32)[:, :25] + c3[...]


def kernel(flat, cu_seqlens, W1, b1, W2, b2, W3, b3, V1, c1, V2, c2, V3, c3):
    xt = flat.reshape(_GRID, _BS, _D).transpose(0, 2, 1)
    vp = jnp.concatenate([
        jnp.pad(V2, ((0, 0), (0, 64))),
        jnp.pad(V3, ((0, 64), (0, 103))),
        jnp.pad(c2[None, :], ((0, 7), (0, 64))),
    ], axis=0).astype(jnp.bfloat16)
    b1r, b2r, b3r = b1[None, :], b2[None, :], b3[None, :]
    full = lambda a: pl.BlockSpec(a.shape, lambda i: (0,) * a.ndim)
    return pl.pallas_call(
        _fused,
        grid=(_GRID,),
        in_specs=[
            pl.BlockSpec(memory_space=pltpu.SMEM),
            pl.BlockSpec((1, _D, _BS), lambda i: (i, 0, 0)),
            full(W1), full(b1r), full(W2), full(b2r), full(W3), full(b3r),
            full(V1), full(c1), full(vp), full(c3),
        ],
        out_specs=pl.BlockSpec((_NSEG, 25), lambda i: (0, 0)),
        out_shape=jax.ShapeDtypeStruct((_NSEG, 25), jnp.float32),
        scratch_shapes=[pltpu.VMEM((128, _NSEG), jnp.float32)],
        compiler_params=pltpu.CompilerParams(
            dimension_semantics=("arbitrary",)),
    )(cu_seqlens, xt, W1, b1r, W2, b2r, W3, b3r,
      V1, c1, vp, c3)


# speedup vs baseline: 1.0431x; 1.0431x over previous
import jax
import jax.numpy as jnp
from jax.experimental import pallas as pl
from jax.experimental.pallas import tpu as pltpu

_TOT = 32768
_D = 3
_NSEG = 16
_BS = 16384
_GRID = _TOT // _BS


def _fused(cu_ref, xt_ref, w1, b1, w2, b2, w3, b3,
           v1, c1, vp, c3, out_ref, acc_ref):
    i = pl.program_id(0)

    @pl.when(i == 0)
    def _init():
        acc_ref[...] = jnp.zeros_like(acc_ref)

    dn = (((0,), (0,)), ((), ()))
    zero = jnp.bfloat16(0.0)
    xt = xt_ref[0].astype(jnp.bfloat16)
    b1c = jnp.transpose(b1[...])
    b2c = jnp.transpose(b2[...])
    b3c = jnp.transpose(b3[...])
    h = jnp.maximum(jax.lax.dot_general(
        w1[...].astype(jnp.bfloat16), xt, dn,
        preferred_element_type=jnp.float32).astype(jnp.bfloat16)
        + b1c.astype(jnp.bfloat16), zero)
    h = jnp.maximum(jax.lax.dot_general(
        w2[...].astype(jnp.bfloat16), h, dn,
        preferred_element_type=jnp.float32).astype(jnp.bfloat16)
        + b2c.astype(jnp.bfloat16), zero)
    h = jnp.maximum(jax.lax.dot_general(
        w3[...].astype(jnp.bfloat16), h, dn,
        preferred_element_type=jnp.float32).astype(jnp.bfloat16)
        + b3c.astype(jnp.bfloat16), zero)

    iota_col = jax.lax.broadcasted_iota(jnp.int32, (_NSEG, 1), 0)
    cu_lo = jnp.zeros((_NSEG, 1), jnp.int32)
    cu_hi = jnp.zeros((_NSEG, 1), jnp.int32)
    for s in range(_NSEG):
        cu_lo = jnp.where(iota_col == s, cu_ref[s], cu_lo)
        cu_hi = jnp.where(iota_col == s, cu_ref[s + 1], cu_hi)
    rows = i * _BS + jax.lax.broadcasted_iota(jnp.int32, (_NSEG, _BS), 1)
    onehot_t = ((rows >= cu_lo) & (rows < cu_hi)).astype(jnp.bfloat16)
    acc_ref[...] += jax.lax.dot_general(
        h, onehot_t, (((1,), (1,)), ((), ())),
        preferred_element_type=jnp.float32)

    @pl.when(i == _GRID - 1)
    def _finish():
        iota_row = jax.lax.broadcasted_iota(jnp.int32, (1, _NSEG), 1)
        cnt = jnp.ones((1, _NSEG), jnp.float32)
        for s in range(_NSEG):
            c = (cu_ref[s + 1] - cu_ref[s]).astype(jnp.float32)
            cnt = jnp.where(iota_row == s, c, cnt)
        pooled = jnp.transpose(acc_ref[...] / jnp.maximum(cnt, 1.0))
        vp_arr = vp[...]
        o = jnp.maximum(jnp.dot(pooled, v1[...],
                                preferred_element_type=jnp.float32) + c1[...], 0.0)
        o = jnp.maximum(jnp.dot(o, vp_arr[0:128],
                                preferred_element_type=jnp.float32)
                        + vp_arr[256:257], 0.0)
        out_ref[...] = jnp.dot(o, vp_arr[128:256],
                               preferred_element_type=jnp.float32)[:, :25] + c3[...]


def kernel(flat, cu_seqlens, W1, b1, W2, b2, W3, b3, V1, c1, V2, c2, V3, c3):
    xt = flat.reshape(_GRID, _BS, _D).transpose(0, 2, 1)
    vp = jnp.concatenate([
        jnp.pad(V2, ((0, 0), (0, 64))),
        jnp.pad(V3, ((0, 64), (0, 103))),
        jnp.pad(c2[None, :], ((0, 7), (0, 64))),
    ], axis=0)
    b1r, b2r, b3r = b1[None, :], b2[None, :], b3[None, :]
    full = lambda a: pl.BlockSpec(a.shape, lambda i: (0,) * a.ndim)
    return pl.pallas_call(
        _fused,
        grid=(_GRID,),
        in_specs=[
            pl.BlockSpec(memory_space=pltpu.SMEM),
            pl.BlockSpec((1, _D, _BS), lambda i: (i, 0, 0)),
            full(W1), full(b1r), full(W2), full(b2r), full(W3), full(b3r),
            full(V1), full(c1), full(vp), full(c3),
        ],
        out_specs=pl.BlockSpec((_NSEG, 25), lambda i: (0, 0)),
        out_shape=jax.ShapeDtypeStruct((_NSEG, 25), jnp.float32),
        scratch_shapes=[pltpu.VMEM((128, _NSEG), jnp.float32)],
        compiler_params=pltpu.CompilerParams(
            dimension_semantics=("arbitrary",)),
    )(cu_seqlens, xt, W1, b1r, W2, b2r, W3, b3r,
      V1, c1, vp, c3)


# speedup vs baseline: 1.0709x; 1.0267x over previous
import jax
import jax.numpy as jnp
from jax.experimental import pallas as pl
from jax.experimental.pallas import tpu as pltpu

_TOT = 32768
_D = 3
_NSEG = 16
_BS = 32768
_GRID = _TOT // _BS


def _fused(cu_ref, xt_ref, w1, b1, w2, b2, w3, b3,
           v1, c1, vp, c3, out_ref, acc_ref):
    i = pl.program_id(0)

    @pl.when(i == 0)
    def _init():
        acc_ref[...] = jnp.zeros_like(acc_ref)

    dn = (((0,), (0,)), ((), ()))
    zero = jnp.bfloat16(0.0)
    xt = xt_ref[0].astype(jnp.bfloat16)
    b1c = jnp.transpose(b1[...])
    b2c = jnp.transpose(b2[...])
    b3c = jnp.transpose(b3[...])
    h = jnp.maximum(jax.lax.dot_general(
        w1[...].astype(jnp.bfloat16), xt, dn,
        preferred_element_type=jnp.float32).astype(jnp.bfloat16)
        + b1c.astype(jnp.bfloat16), zero)
    h = jnp.maximum(jax.lax.dot_general(
        w2[...].astype(jnp.bfloat16), h, dn,
        preferred_element_type=jnp.float32).astype(jnp.bfloat16)
        + b2c.astype(jnp.bfloat16), zero)
    h = jnp.maximum(jax.lax.dot_general(
        w3[...].astype(jnp.bfloat16), h, dn,
        preferred_element_type=jnp.float32).astype(jnp.bfloat16)
        + b3c.astype(jnp.bfloat16), zero)

    iota_col = jax.lax.broadcasted_iota(jnp.int32, (_NSEG, 1), 0)
    cu_lo = jnp.zeros((_NSEG, 1), jnp.int32)
    cu_hi = jnp.zeros((_NSEG, 1), jnp.int32)
    for s in range(_NSEG):
        cu_lo = jnp.where(iota_col == s, cu_ref[s], cu_lo)
        cu_hi = jnp.where(iota_col == s, cu_ref[s + 1], cu_hi)
    rows = i * _BS + jax.lax.broadcasted_iota(jnp.int32, (_NSEG, _BS), 1)
    onehot_t = ((rows >= cu_lo) & (rows < cu_hi)).astype(jnp.bfloat16)
    acc_ref[...] += jax.lax.dot_general(
        h, onehot_t, (((1,), (1,)), ((), ())),
        preferred_element_type=jnp.float32)

    @pl.when(i == _GRID - 1)
    def _finish():
        iota_row = jax.lax.broadcasted_iota(jnp.int32, (1, _NSEG), 1)
        cnt = jnp.ones((1, _NSEG), jnp.float32)
        for s in range(_NSEG):
            c = (cu_ref[s + 1] - cu_ref[s]).astype(jnp.float32)
            cnt = jnp.where(iota_row == s, c, cnt)
        pooled = jnp.transpose(acc_ref[...] / jnp.maximum(cnt, 1.0))
        vp_arr = vp[...]
        o = jnp.maximum(jnp.dot(pooled, v1[...],
                                preferred_element_type=jnp.float32) + c1[...], 0.0)
        o = jnp.maximum(jnp.dot(o, vp_arr[0:128],
                                preferred_element_type=jnp.float32)
                        + vp_arr[256:257], 0.0)
        out_ref[...] = jnp.dot(o, vp_arr[128:256],
                               preferred_element_type=jnp.float32)[:, :25] + c3[...]


def kernel(flat, cu_seqlens, W1, b1, W2, b2, W3, b3, V1, c1, V2, c2, V3, c3):
    xt = flat.reshape(_GRID, _BS, _D).transpose(0, 2, 1)
    vp = jnp.concatenate([
        jnp.pad(V2, ((0, 0), (0, 64))),
        jnp.pad(V3, ((0, 64), (0, 103))),
        jnp.pad(c2[None, :], ((0, 7), (0, 64))),
    ], axis=0)
    b1r, b2r, b3r = b1[None, :], b2[None, :], b3[None, :]
    full = lambda a: pl.BlockSpec(a.shape, lambda i: (0,) * a.ndim)
    return pl.pallas_call(
        _fused,
        grid=(_GRID,),
        in_specs=[
            pl.BlockSpec(memory_space=pltpu.SMEM),
            pl.BlockSpec((1, _D, _BS), lambda i: (i, 0, 0)),
            full(W1), full(b1r), full(W2), full(b2r), full(W3), full(b3r),
            full(V1), full(c1), full(vp), full(c3),
        ],
        out_specs=pl.BlockSpec((_NSEG, 25), lambda i: (0, 0)),
        out_shape=jax.ShapeDtypeStruct((_NSEG, 25), jnp.float32),
        scratch_shapes=[pltpu.VMEM((128, _NSEG), jnp.float32)],
        compiler_params=pltpu.CompilerParams(
            dimension_semantics=("arbitrary",)),
    )(cu_seqlens, xt, W1, b1r, W2, b2r, W3, b3r,
      V1, c1, vp, c3)
